# Initial kernel scaffold; baseline (speedup 1.0000x reference)
import jax
import jax.numpy as jnp
from jax.experimental import pallas as pl


def kernel(x, pos, batch, num_graphs, l1_w, l1_b, bn1_g, bn1_b, l2_w, l2_b, bn2_g, bn2_b, l3_w, l3_b, bn3_g, bn3_b, cva_w, cva_b, bn4_g, bn4_b, cvb_w, cvb_b, bn5_g, bn5_b, cvc_w, cvc_b, lw, lb, c1_w, c1_b, bnc1_g, bnc1_b, c2_w, c2_b, bnc2_g, bnc2_b):
    raise NotImplementedError("write your pallas kernel here")


# speedup vs baseline: 2.1735x; 2.1735x over previous
import jax
import jax.numpy as jnp
from jax.experimental import pallas as pl
from jax.experimental.pallas import tpu as pltpu

N = 8192
B = 16
D_POS = 30
D_IN = 40
K = 4
C_DELTA = 10
ROW_BLK = 256
D_TAB = D_POS + D_IN


def _knn_gather_kernel(tb_blk, tb_full, posT, brow, bcol, g_out):
    pos_blk = tb_blk[:, :D_POS]
    posTv = posT[...]
    sqr = jnp.sum(pos_blk * pos_blk, axis=1, keepdims=True)
    sqc = jnp.sum(posTv * posTv, axis=0, keepdims=True)
    cross = jax.lax.dot_general(
        pos_blk.astype(jnp.bfloat16), posTv.astype(jnp.bfloat16),
        (((1,), (0,)), ((), ())),
        preferred_element_type=jnp.float32)
    d2 = sqr + sqc - 2.0 * cross
    mask = bcol[...] != brow[...]
    d2 = jnp.where(mask, 1e30, d2)

    iota = jax.lax.broadcasted_iota(jnp.int32, (ROW_BLK, N), 1)
    tab = tb_full[...]
    parts = []
    for _ in range(K):
        vmin = jnp.min(d2, axis=1, keepdims=True)
        idx = jnp.min(jnp.where(d2 == vmin, iota, N), axis=1, keepdims=True)
        oh = (iota == idx)
        gathered = jax.lax.dot_general(
            oh.astype(jnp.float32), tab, (((1,), (0,)), ((), ())),
            precision=jax.lax.Precision.HIGHEST,
            preferred_element_type=jnp.float32)
        parts.append(gathered)
        d2 = jnp.where(oh, 3e38, d2)
    g_out[...] = jnp.concatenate(parts, axis=1)


def _elu(v):
    return jnp.where(v > 0, v, jnp.exp(jnp.minimum(v, 0.0)) - 1.0)


CH = 512
NCH = N // CH


def _mlp_kernel(g, pos, brow,
                l1_wT, l1_b, bn1_g, bn1_b, l2_wT, l2_b, bn2_g, bn2_b,
                l3_wT, l3_b, bn3_g, bn3_b, wa, cva_b, bn4_g, bn4_b,
                wb, cvb_b, bn5_g, bn5_b, cvc_r, cvcb_r, lw_r, lb,
                c1_wT, c1_b, bnc1_g, bnc1_b, c2_wT, c2_b, bnc2_g, bnc2_b,
                out, hs, ts):
    eps = 1e-5

    def dot(a, b):
        return jax.lax.dot_general(a.astype(jnp.bfloat16),
                                   b.astype(jnp.bfloat16),
                                   (((1,), (0,)), ((), ())),
                                   preferred_element_type=jnp.float32)

    def xdot(a, b):
        return jax.lax.dot_general(a, b, (((1,), (0,)), ((), ())),
                                   precision=jax.lax.Precision.HIGHEST,
                                   preferred_element_type=jnp.float32)

    def b16(v):
        return v.astype(jnp.bfloat16).astype(jnp.float32)

    l1w = l1_wT[...]; l1bv = l1_b[...]
    l2w = l2_wT[...]; l2bv = l2_b[...]
    l3w = l3_wT[...]; l3bv = l3_b[...]
    wav = wa[...]; cvabv = cva_b[...]
    wbv = wb[...]; cvbbv = cvb_b[...]
    cvc = cvc_r[...]; cvcb = cvcb_r[...]
    lwv = lw_r[...]; lbv = lb[...]

    def stats_pair(arr):
        return (jnp.sum(arr, axis=0, keepdims=True),
                jnp.sum(arr * arr, axis=0, keepdims=True))

    def finalize(s, q, rows, gam, bet):
        m = s / rows
        v = q / rows - m * m
        inv = gam[...] / jnp.sqrt(v + eps)
        return inv, bet[...] - m * inv

    def fold4(v40):
        return sum(v40[:, k * C_DELTA:(k + 1) * C_DELTA] for k in range(K))

    z40 = jnp.zeros((1, K * C_DELTA), jnp.float32)
    z16 = jnp.zeros((1, K * K), jnp.float32)

    def ph1(i, c):
        s40, q40, s16, q16 = c
        sl = pl.ds(i * CH, CH)
        gch = g[sl, :]
        pch = pos[sl, :]
        rel = [gch[:, k * D_TAB:k * D_TAB + D_POS] - pch for k in range(K)]
        hcat = jnp.concatenate(
            [_elu(dot(r, l1w) + l1bv) for r in rel], axis=1)
        t1 = _elu(dot(jnp.concatenate(rel, axis=1), l3w) + l3bv)
        hs[sl, :] = hcat
        ts[sl, :] = t1
        a, b2 = stats_pair(hcat)
        c2, d2 = stats_pair(t1)
        return (s40 + a, q40 + b2, s16 + c2, q16 + d2)

    s40, q40, s16, q16 = jax.lax.fori_loop(
        0, NCH, ph1, (z40, z40, z16, z16))
    inv1, sh1 = finalize(fold4(s40), fold4(q40), K * N, bn1_g, bn1_b)
    inv1 = jnp.concatenate([inv1] * K, axis=1)
    sh1 = jnp.concatenate([sh1] * K, axis=1)
    inv3, sh3 = finalize(s16, q16, N, bn3_g, bn3_b)

    def ph2(i, c):
        s40, q40, s16, q16 = c
        sl = pl.ds(i * CH, CH)
        h1 = hs[sl, :] * inv1 + sh1
        hcat = jnp.concatenate(
            [_elu(dot(h1[:, k * C_DELTA:(k + 1) * C_DELTA], l2w) + l2bv)
             for k in range(K)], axis=1)
        t1 = ts[sl, :] * inv3 + sh3
        t2 = _elu(dot(t1, wav) + cvabv)
        hs[sl, :] = hcat
        ts[sl, :] = t2
        a, b2 = stats_pair(hcat)
        c2, d2 = stats_pair(t2)
        return (s40 + a, q40 + b2, s16 + c2, q16 + d2)

    s40, q40, s16, q16 = jax.lax.fori_loop(
        0, NCH, ph2, (z40, z40, z16, z16))
    inv2, sh2 = finalize(fold4(s40), fold4(q40), K * N, bn2_g, bn2_b)
    inv2 = jnp.concatenate([inv2] * K, axis=1)
    sh2 = jnp.concatenate([sh2] * K, axis=1)
    inv4, sh4 = finalize(s16, q16, N, bn4_g, bn4_b)

    def ph3(i, c):
        s16, q16 = c
        sl = pl.ds(i * CH, CH)
        t2 = ts[sl, :] * inv4 + sh4
        t3 = dot(t2, wbv) + cvbbv
        ts[sl, :] = t3
        c2, d2 = stats_pair(t3)
        return (s16 + c2, q16 + d2)

    s16, q16 = jax.lax.fori_loop(0, NCH, ph3, (z16, z16))
    inv5, sh5 = finalize(s16, q16, N, bn5_g, bn5_b)

    def ph4(i, c):
        sums, cnt = c
        sl = pl.ds(i * CH, CH)
        gch = g[sl, :]
        hb = hs[sl, :] * inv2 + sh2
        t = ts[sl, :] * inv5 + sh5
        sk = [b16(jnp.concatenate(
            [hb[:, k * C_DELTA:(k + 1) * C_DELTA],
             gch[:, k * D_TAB + D_POS:(k + 1) * D_TAB]], axis=1))
            for k in range(K)]
        tb = b16(t)
        xt = [sum(sk[k] * tb[:, K * k + j:K * k + j + 1] for k in range(K))
              for j in range(K)]
        cvc16 = b16(cvc)
        ych = lbv
        for o in range(3):
            out_o = cvcb[o:o + 1, :]
            for j in range(K):
                out_o = out_o + b16(xt[j]) * cvc16[o * K + j:o * K + j + 1, :]
            ych = ych + dot(out_o, lwv[o * 50:(o + 1) * 50, :])
        bch = brow[:, sl]
        iot = jax.lax.broadcasted_iota(
            jnp.int32, (B, CH), 0).astype(jnp.float32)
        seg = (iot == bch).astype(jnp.float32)
        return (sums + xdot(seg, ych),
                cnt + jnp.sum(seg, axis=1, keepdims=True))

    sums, cnt = jax.lax.fori_loop(
        0, NCH, ph4,
        (jnp.zeros((B, 128), jnp.float32), jnp.zeros((B, 1), jnp.float32)))

    pooled = sums / jnp.maximum(cnt, 1.0)
    z = jnp.maximum(dot(pooled, c1_wT[...]) + c1_b[...], 0.0)
    s1, q1 = stats_pair(z)
    i1, h1 = finalize(s1, q1, B, bnc1_g, bnc1_b)
    z = z * i1 + h1
    z = jnp.maximum(dot(z, c2_wT[...]) + c2_b[...], 0.0)
    s2, q2 = stats_pair(z)
    i2, h2 = finalize(s2, q2, B, bnc2_g, bnc2_b)
    z = z * i2 + h2
    out[...] = 1.0 / (1.0 + jnp.exp(-z))


def kernel(x, pos, batch, num_graphs, l1_w, l1_b, bn1_g, bn1_b, l2_w, l2_b,
           bn2_g, bn2_b, l3_w, l3_b, bn3_g, bn3_b, cva_w, cva_b, bn4_g, bn4_b,
           cvb_w, cvb_b, bn5_g, bn5_b, cvc_w, cvc_b, lw, lb, c1_w, c1_b,
           bnc1_g, bnc1_b, c2_w, c2_b, bnc2_g, bnc2_b):
    del num_graphs
    table = jnp.concatenate([pos, x], axis=1)
    posT = pos.T
    bf = batch.astype(jnp.float32)
    brow = bf.reshape(1, N)
    bcol = bf.reshape(N, 1)

    nblk = N // ROW_BLK
    g = pl.pallas_call(
        _knn_gather_kernel,
        grid=(nblk,),
        in_specs=[
            pl.BlockSpec((ROW_BLK, D_TAB), lambda i: (i, 0)),
            pl.BlockSpec((N, D_TAB), lambda i: (0, 0)),
            pl.BlockSpec((D_POS, N), lambda i: (0, 0)),
            pl.BlockSpec((1, N), lambda i: (0, 0)),
            pl.BlockSpec((ROW_BLK, 1), lambda i: (i, 0)),
        ],
        out_specs=pl.BlockSpec((ROW_BLK, K * D_TAB), lambda i: (i, 0)),
        out_shape=jax.ShapeDtypeStruct((N, K * D_TAB), jnp.float32),
    )(table, table, posT, brow, bcol)

    eye = jnp.eye(K, dtype=jnp.float32)
    wa = (jnp.einsum('gok,gh->gkho', cva_w, eye)).reshape(K * K, K * K)
    wb = (jnp.einsum('gok,gh->gkho', cvb_w, eye)).reshape(K * K, K * K)
    cvc_r = jnp.transpose(cvc_w, (1, 2, 0)).reshape(3 * K, 50)
    cvcb_r = cvc_b.reshape(50, 3).T
    lw_r = jnp.transpose(lw.reshape(128, 50, 3), (2, 1, 0)).reshape(150, 128)

    args = [
        g, pos, brow,
        l1_w.T, l1_b.reshape(1, -1), bn1_g.reshape(1, -1), bn1_b.reshape(1, -1),
        l2_w.T, l2_b.reshape(1, -1), bn2_g.reshape(1, -1), bn2_b.reshape(1, -1),
        l3_w.T, l3_b.reshape(1, -1), bn3_g.reshape(1, -1), bn3_b.reshape(1, -1),
        wa, cva_b.reshape(1, -1), bn4_g.reshape(1, -1), bn4_b.reshape(1, -1),
        wb, cvb_b.reshape(1, -1), bn5_g.reshape(1, -1), bn5_b.reshape(1, -1),
        cvc_r, cvcb_r, lw_r, lb.reshape(1, -1),
        c1_w.T, c1_b.reshape(1, -1), bnc1_g.reshape(1, -1), bnc1_b.reshape(1, -1),
        c2_w.T, c2_b.reshape(1, -1), bnc2_g.reshape(1, -1), bnc2_b.reshape(1, -1),
    ]
    out = pl.pallas_call(
        _mlp_kernel,
        out_shape=jax.ShapeDtypeStruct((B, 1), jnp.float32),
        scratch_shapes=[
            pltpu.VMEM((N, K * C_DELTA), jnp.float32),
            pltpu.VMEM((N, K * K), jnp.float32),
        ],
    )(*args)
    return out[:, 0]


# speedup vs baseline: 6.4390x; 2.9625x over previous
import jax
import jax.numpy as jnp
from jax.experimental import pallas as pl
from jax.experimental.pallas import tpu as pltpu

N = 8192
B = 16
D_POS = 30
D_IN = 40
K = 4
C_DELTA = 10
ROW_BLK = 256
D_TAB = D_POS + D_IN


def _knn_gather_kernel(tb_blk, tb_full, posT, brow, bcol, g_out):
    pos_blk = tb_blk[:, :D_POS]
    posTv = posT[...]
    sqr = jnp.sum(pos_blk * pos_blk, axis=1, keepdims=True)
    sqc = jnp.sum(posTv * posTv, axis=0, keepdims=True)
    cross = jax.lax.dot_general(
        pos_blk.astype(jnp.bfloat16), posTv.astype(jnp.bfloat16),
        (((1,), (0,)), ((), ())),
        preferred_element_type=jnp.float32)
    d2 = sqr + sqc - 2.0 * cross
    mask = bcol[...] != brow[...]
    d2 = jnp.where(mask, 1e30, d2)

    iota = jax.lax.broadcasted_iota(jnp.int32, (ROW_BLK, N), 1)
    tab3 = tb_full[...]
    parts = []
    for _ in range(K):
        vmin = jnp.min(d2, axis=1, keepdims=True)
        idx = jnp.min(jnp.where(d2 == vmin, iota, N), axis=1, keepdims=True)
        oh = (iota == idx)
        g3 = jax.lax.dot_general(
            oh.astype(jnp.bfloat16), tab3, (((1,), (0,)), ((), ())),
            preferred_element_type=jnp.float32)
        parts.append(g3[:, :D_TAB] + g3[:, D_TAB:2 * D_TAB] + g3[:, 2 * D_TAB:])
        d2 = jnp.where(oh, 3e38, d2)
    g_out[...] = jnp.concatenate(parts, axis=1)


def _elu(v):
    return jnp.where(v > 0, v, jnp.exp(jnp.minimum(v, 0.0)) - 1.0)


CH = 512
NCH = N // CH


def _mlp_kernel(g, pos, brow,
                l1_wT, l1_b, bn1_g, bn1_b, l2_wT, l2_b, bn2_g, bn2_b,
                l3_wT, l3_b, bn3_g, bn3_b, wa, cva_b, bn4_g, bn4_b,
                wb, cvb_b, bn5_g, bn5_b, cvc_r, cvcb_r, lw_r, lb,
                c1_wT, c1_b, bnc1_g, bnc1_b, c2_wT, c2_b, bnc2_g, bnc2_b,
                out, hs, ts):
    eps = 1e-5

    def dot(a, b):
        return jax.lax.dot_general(a.astype(jnp.bfloat16),
                                   b.astype(jnp.bfloat16),
                                   (((1,), (0,)), ((), ())),
                                   preferred_element_type=jnp.float32)

    def xdot(a, b):
        return jax.lax.dot_general(a, b, (((1,), (0,)), ((), ())),
                                   precision=jax.lax.Precision.HIGHEST,
                                   preferred_element_type=jnp.float32)

    def b16(v):
        return v.astype(jnp.bfloat16).astype(jnp.float32)

    l1w = l1_wT[...]; l1bv = l1_b[...]
    l2w = l2_wT[...]; l2bv = l2_b[...]
    l3w = l3_wT[...]; l3bv = l3_b[...]
    wav = wa[...]; cvabv = cva_b[...]
    wbv = wb[...]; cvbbv = cvb_b[...]
    cvc = cvc_r[...]; cvcb = cvcb_r[...]
    lwv = lw_r[...]; lbv = lb[...]

    def stats_pair(arr):
        return (jnp.sum(arr, axis=0, keepdims=True),
                jnp.sum(arr * arr, axis=0, keepdims=True))

    def finalize(s, q, rows, gam, bet):
        m = s / rows
        v = q / rows - m * m
        inv = gam[...] / jnp.sqrt(v + eps)
        return inv, bet[...] - m * inv

    def fold4(v40):
        return sum(v40[:, k * C_DELTA:(k + 1) * C_DELTA] for k in range(K))

    z40 = jnp.zeros((1, K * C_DELTA), jnp.float32)
    z16 = jnp.zeros((1, K * K), jnp.float32)

    def ph1(i, c):
        s40, q40, s16, q16 = c
        sl = pl.ds(i * CH, CH)
        gch = g[sl, :]
        pch = pos[sl, :]
        rel = [gch[:, k * D_TAB:k * D_TAB + D_POS] - pch for k in range(K)]
        hcat = jnp.concatenate(
            [_elu(dot(r, l1w) + l1bv) for r in rel], axis=1)
        t1 = _elu(dot(jnp.concatenate(rel, axis=1), l3w) + l3bv)
        hs[sl, :] = hcat
        ts[sl, :] = t1
        a, b2 = stats_pair(hcat)
        c2, d2 = stats_pair(t1)
        return (s40 + a, q40 + b2, s16 + c2, q16 + d2)

    s40, q40, s16, q16 = jax.lax.fori_loop(
        0, NCH, ph1, (z40, z40, z16, z16))
    inv1, sh1 = finalize(fold4(s40), fold4(q40), K * N, bn1_g, bn1_b)
    inv1 = jnp.concatenate([inv1] * K, axis=1)
    sh1 = jnp.concatenate([sh1] * K, axis=1)
    inv3, sh3 = finalize(s16, q16, N, bn3_g, bn3_b)

    def ph2(i, c):
        s40, q40, s16, q16 = c
        sl = pl.ds(i * CH, CH)
        h1 = hs[sl, :] * inv1 + sh1
        hcat = jnp.concatenate(
            [_elu(dot(h1[:, k * C_DELTA:(k + 1) * C_DELTA], l2w) + l2bv)
             for k in range(K)], axis=1)
        t1 = ts[sl, :] * inv3 + sh3
        t2 = _elu(dot(t1, wav) + cvabv)
        hs[sl, :] = hcat
        ts[sl, :] = t2
        a, b2 = stats_pair(hcat)
        c2, d2 = stats_pair(t2)
        return (s40 + a, q40 + b2, s16 + c2, q16 + d2)

    s40, q40, s16, q16 = jax.lax.fori_loop(
        0, NCH, ph2, (z40, z40, z16, z16))
    inv2, sh2 = finalize(fold4(s40), fold4(q40), K * N, bn2_g, bn2_b)
    inv2 = jnp.concatenate([inv2] * K, axis=1)
    sh2 = jnp.concatenate([sh2] * K, axis=1)
    inv4, sh4 = finalize(s16, q16, N, bn4_g, bn4_b)

    def ph3(i, c):
        s16, q16 = c
        sl = pl.ds(i * CH, CH)
        t2 = ts[sl, :] * inv4 + sh4
        t3 = dot(t2, wbv) + cvbbv
        ts[sl, :] = t3
        c2, d2 = stats_pair(t3)
        return (s16 + c2, q16 + d2)

    s16, q16 = jax.lax.fori_loop(0, NCH, ph3, (z16, z16))
    inv5, sh5 = finalize(s16, q16, N, bn5_g, bn5_b)

    def ph4(i, c):
        sums, cnt = c
        sl = pl.ds(i * CH, CH)
        gch = g[sl, :]
        hb = hs[sl, :] * inv2 + sh2
        t = ts[sl, :] * inv5 + sh5
        sk = [b16(jnp.concatenate(
            [hb[:, k * C_DELTA:(k + 1) * C_DELTA],
             gch[:, k * D_TAB + D_POS:(k + 1) * D_TAB]], axis=1))
            for k in range(K)]
        tb = b16(t)
        xt = [sum(sk[k] * tb[:, K * k + j:K * k + j + 1] for k in range(K))
              for j in range(K)]
        cvc16 = b16(cvc)
        ych = lbv
        for o in range(3):
            out_o = cvcb[o:o + 1, :]
            for j in range(K):
                out_o = out_o + b16(xt[j]) * cvc16[o * K + j:o * K + j + 1, :]
            ych = ych + dot(out_o, lwv[o * 50:(o + 1) * 50, :])
        bch = brow[:, sl]
        iot = jax.lax.broadcasted_iota(
            jnp.int32, (B, CH), 0).astype(jnp.float32)
        seg = (iot == bch).astype(jnp.float32)
        return (sums + xdot(seg, ych),
                cnt + jnp.sum(seg, axis=1, keepdims=True))

    sums, cnt = jax.lax.fori_loop(
        0, NCH, ph4,
        (jnp.zeros((B, 128), jnp.float32), jnp.zeros((B, 1), jnp.float32)))

    pooled = sums / jnp.maximum(cnt, 1.0)
    z = jnp.maximum(dot(pooled, c1_wT[...]) + c1_b[...], 0.0)
    s1, q1 = stats_pair(z)
    i1, h1 = finalize(s1, q1, B, bnc1_g, bnc1_b)
    z = z * i1 + h1
    z = jnp.maximum(dot(z, c2_wT[...]) + c2_b[...], 0.0)
    s2, q2 = stats_pair(z)
    i2, h2 = finalize(s2, q2, B, bnc2_g, bnc2_b)
    z = z * i2 + h2
    out[...] = 1.0 / (1.0 + jnp.exp(-z))


def kernel(x, pos, batch, num_graphs, l1_w, l1_b, bn1_g, bn1_b, l2_w, l2_b,
           bn2_g, bn2_b, l3_w, l3_b, bn3_g, bn3_b, cva_w, cva_b, bn4_g, bn4_b,
           cvb_w, cvb_b, bn5_g, bn5_b, cvc_w, cvc_b, lw, lb, c1_w, c1_b,
           bnc1_g, bnc1_b, c2_w, c2_b, bnc2_g, bnc2_b):
    del num_graphs
    table = jnp.concatenate([pos, x], axis=1)
    t1 = table.astype(jnp.bfloat16)
    r1 = table - t1.astype(jnp.float32)
    t2 = r1.astype(jnp.bfloat16)
    t3 = (r1 - t2.astype(jnp.float32)).astype(jnp.bfloat16)
    tab3 = jnp.concatenate([t1, t2, t3], axis=1)
    posT = pos.T
    bf = batch.astype(jnp.float32)
    brow = bf.reshape(1, N)
    bcol = bf.reshape(N, 1)

    nblk = N // ROW_BLK
    g = pl.pallas_call(
        _knn_gather_kernel,
        grid=(nblk,),
        in_specs=[
            pl.BlockSpec((ROW_BLK, D_TAB), lambda i: (i, 0)),
            pl.BlockSpec((N, 3 * D_TAB), lambda i: (0, 0)),
            pl.BlockSpec((D_POS, N), lambda i: (0, 0)),
            pl.BlockSpec((1, N), lambda i: (0, 0)),
            pl.BlockSpec((ROW_BLK, 1), lambda i: (i, 0)),
        ],
        out_specs=pl.BlockSpec((ROW_BLK, K * D_TAB), lambda i: (i, 0)),
        out_shape=jax.ShapeDtypeStruct((N, K * D_TAB), jnp.float32),
    )(table, tab3, posT, brow, bcol)

    eye = jnp.eye(K, dtype=jnp.float32)
    wa = (jnp.einsum('gok,gh->gkho', cva_w, eye)).reshape(K * K, K * K)
    wb = (jnp.einsum('gok,gh->gkho', cvb_w, eye)).reshape(K * K, K * K)
    cvc_r = jnp.transpose(cvc_w, (1, 2, 0)).reshape(3 * K, 50)
    cvcb_r = cvc_b.reshape(50, 3).T
    lw_r = jnp.transpose(lw.reshape(128, 50, 3), (2, 1, 0)).reshape(150, 128)

    args = [
        g, pos, brow,
        l1_w.T, l1_b.reshape(1, -1), bn1_g.reshape(1, -1), bn1_b.reshape(1, -1),
        l2_w.T, l2_b.reshape(1, -1), bn2_g.reshape(1, -1), bn2_b.reshape(1, -1),
        l3_w.T, l3_b.reshape(1, -1), bn3_g.reshape(1, -1), bn3_b.reshape(1, -1),
        wa, cva_b.reshape(1, -1), bn4_g.reshape(1, -1), bn4_b.reshape(1, -1),
        wb, cvb_b.reshape(1, -1), bn5_g.reshape(1, -1), bn5_b.reshape(1, -1),
        cvc_r, cvcb_r, lw_r, lb.reshape(1, -1),
        c1_w.T, c1_b.reshape(1, -1), bnc1_g.reshape(1, -1), bnc1_b.reshape(1, -1),
        c2_w.T, c2_b.reshape(1, -1), bnc2_g.reshape(1, -1), bnc2_b.reshape(1, -1),
    ]
    out = pl.pallas_call(
        _mlp_kernel,
        out_shape=jax.ShapeDtypeStruct((B, 1), jnp.float32),
        scratch_shapes=[
            pltpu.VMEM((N, K * C_DELTA), jnp.float32),
            pltpu.VMEM((N, K * K), jnp.float32),
        ],
    )(*args)
    return out[:, 0]


# speedup vs baseline: 7.9724x; 1.2381x over previous
import functools

import jax
import jax.numpy as jnp
from jax.experimental import pallas as pl
from jax.experimental.pallas import tpu as pltpu
from jax.experimental.pallas import tpu_sc as plsc

N = 8192
B = 16
D_POS = 30
D_IN = 40
K = 4
C_DELTA = 10
ROW_BLK = 256
D_TAB = D_POS + D_IN


def _knn_gather_kernel(pos_blk_ref, posT, brow, bcol, idx_out):
    pos_blk = pos_blk_ref[...]
    posTv = posT[...]
    sqr = jnp.sum(pos_blk * pos_blk, axis=1, keepdims=True)
    sqc = jnp.sum(posTv * posTv, axis=0, keepdims=True)
    cross = jax.lax.dot_general(
        pos_blk.astype(jnp.bfloat16), posTv.astype(jnp.bfloat16),
        (((1,), (0,)), ((), ())),
        preferred_element_type=jnp.float32)
    d2 = sqr + sqc - 2.0 * cross
    mask = bcol[...] != brow[...]
    d2 = jnp.where(mask, 1e30, d2)

    iota = jax.lax.broadcasted_iota(jnp.int32, (ROW_BLK, N), 1)
    parts = []
    for _ in range(K):
        vmin = jnp.min(d2, axis=1, keepdims=True)
        idx = jnp.min(jnp.where(d2 == vmin, iota, N), axis=1, keepdims=True)
        parts.append(idx.astype(jnp.float32))
        d2 = jnp.where(iota == idx, 3e38, d2)
    idx_out[...] = jnp.concatenate(parts, axis=1)


D_PAD = 128
SC_CHUNK = 512


def _sc_gather(table_pad, idx_flat):
    info = plsc.get_sparse_core_info()
    nw = info.num_cores * info.num_subcores
    bpw = (K * N) // nw
    nchunk = bpw // SC_CHUNK
    mesh = plsc.VectorSubcoreMesh(core_axis_name="c", subcore_axis_name="s")

    @functools.partial(
        pl.kernel, mesh=mesh,
        out_type=jax.ShapeDtypeStruct((K * N, D_PAD), jnp.float32),
        scratch_types=[
            pltpu.VMEM((SC_CHUNK,), jnp.int32),
            pltpu.VMEM((SC_CHUNK, D_PAD), jnp.float32),
            pltpu.SemaphoreType.DMA,
        ],
    )
    def gk(table_hbm, idx_hbm, out_hbm, idx_v, rows_v, sem):
        wid = jax.lax.axis_index("s") * info.num_cores + jax.lax.axis_index("c")
        base = wid * bpw
        for c in range(nchunk):
            off = base + c * SC_CHUNK
            pltpu.sync_copy(idx_hbm.at[pl.ds(off, SC_CHUNK)], idx_v)
            pltpu.async_copy(table_hbm.at[idx_v], rows_v, sem).wait()
            pltpu.sync_copy(rows_v, out_hbm.at[pl.ds(off, SC_CHUNK)])

    return gk(table_pad, idx_flat)


def _elu(v):
    return jnp.where(v > 0, v, jnp.exp(jnp.minimum(v, 0.0)) - 1.0)


CH = 512
NCH = N // CH


def _mlp_kernel(g, pos, brow,
                l1_wT, l1_b, bn1_g, bn1_b, l2_wT, l2_b, bn2_g, bn2_b,
                l3_wT, l3_b, bn3_g, bn3_b, wa, cva_b, bn4_g, bn4_b,
                wb, cvb_b, bn5_g, bn5_b, cvc_r, cvcb_r, lw_r, lb,
                c1_wT, c1_b, bnc1_g, bnc1_b, c2_wT, c2_b, bnc2_g, bnc2_b,
                out, hs, ts):
    eps = 1e-5

    def dot(a, b):
        return jax.lax.dot_general(a.astype(jnp.bfloat16),
                                   b.astype(jnp.bfloat16),
                                   (((1,), (0,)), ((), ())),
                                   preferred_element_type=jnp.float32)

    def xdot(a, b):
        return jax.lax.dot_general(a, b, (((1,), (0,)), ((), ())),
                                   precision=jax.lax.Precision.HIGHEST,
                                   preferred_element_type=jnp.float32)

    def b16(v):
        return v.astype(jnp.bfloat16).astype(jnp.float32)

    l1w = l1_wT[...]; l1bv = l1_b[...]
    l2w = l2_wT[...]; l2bv = l2_b[...]
    l3w = l3_wT[...]; l3bv = l3_b[...]
    wav = wa[...]; cvabv = cva_b[...]
    wbv = wb[...]; cvbbv = cvb_b[...]
    cvc = cvc_r[...]; cvcb = cvcb_r[...]
    lwv = lw_r[...]; lbv = lb[...]

    def stats_pair(arr):
        return (jnp.sum(arr, axis=0, keepdims=True),
                jnp.sum(arr * arr, axis=0, keepdims=True))

    def finalize(s, q, rows, gam, bet):
        m = s / rows
        v = q / rows - m * m
        inv = gam[...] / jnp.sqrt(v + eps)
        return inv, bet[...] - m * inv

    def fold4(v40):
        return sum(v40[:, k * C_DELTA:(k + 1) * C_DELTA] for k in range(K))

    z40 = jnp.zeros((1, K * C_DELTA), jnp.float32)
    z16 = jnp.zeros((1, K * K), jnp.float32)

    def ph1(i, c):
        s40, q40, s16, q16 = c
        sl = pl.ds(i * CH, CH)
        pch = pos[sl, :]
        gks = [g[pl.ds(k * N + i * CH, CH), :] for k in range(K)]
        rel = [gk[:, :D_POS] - pch for gk in gks]
        hcat = jnp.concatenate(
            [_elu(dot(r, l1w) + l1bv) for r in rel], axis=1)
        t1 = _elu(dot(jnp.concatenate(rel, axis=1), l3w) + l3bv)
        hs[sl, :] = hcat
        ts[sl, :] = t1
        a, b2 = stats_pair(hcat)
        c2, d2 = stats_pair(t1)
        return (s40 + a, q40 + b2, s16 + c2, q16 + d2)

    s40, q40, s16, q16 = jax.lax.fori_loop(
        0, NCH, ph1, (z40, z40, z16, z16))
    inv1, sh1 = finalize(fold4(s40), fold4(q40), K * N, bn1_g, bn1_b)
    inv1 = jnp.concatenate([inv1] * K, axis=1)
    sh1 = jnp.concatenate([sh1] * K, axis=1)
    inv3, sh3 = finalize(s16, q16, N, bn3_g, bn3_b)

    def ph2(i, c):
        s40, q40, s16, q16 = c
        sl = pl.ds(i * CH, CH)
        h1 = hs[sl, :] * inv1 + sh1
        hcat = jnp.concatenate(
            [_elu(dot(h1[:, k * C_DELTA:(k + 1) * C_DELTA], l2w) + l2bv)
             for k in range(K)], axis=1)
        t1 = ts[sl, :] * inv3 + sh3
        t2 = _elu(dot(t1, wav) + cvabv)
        hs[sl, :] = hcat
        ts[sl, :] = t2
        a, b2 = stats_pair(hcat)
        c2, d2 = stats_pair(t2)
        return (s40 + a, q40 + b2, s16 + c2, q16 + d2)

    s40, q40, s16, q16 = jax.lax.fori_loop(
        0, NCH, ph2, (z40, z40, z16, z16))
    inv2, sh2 = finalize(fold4(s40), fold4(q40), K * N, bn2_g, bn2_b)
    inv2 = jnp.concatenate([inv2] * K, axis=1)
    sh2 = jnp.concatenate([sh2] * K, axis=1)
    inv4, sh4 = finalize(s16, q16, N, bn4_g, bn4_b)

    def ph3(i, c):
        s16, q16 = c
        sl = pl.ds(i * CH, CH)
        t2 = ts[sl, :] * inv4 + sh4
        t3 = dot(t2, wbv) + cvbbv
        ts[sl, :] = t3
        c2, d2 = stats_pair(t3)
        return (s16 + c2, q16 + d2)

    s16, q16 = jax.lax.fori_loop(0, NCH, ph3, (z16, z16))
    inv5, sh5 = finalize(s16, q16, N, bn5_g, bn5_b)

    def ph4(i, c):
        sums, cnt = c
        sl = pl.ds(i * CH, CH)
        hb = hs[sl, :] * inv2 + sh2
        t = ts[sl, :] * inv5 + sh5
        gks = [g[pl.ds(k * N + i * CH, CH), :] for k in range(K)]
        sk = [b16(jnp.concatenate(
            [hb[:, k * C_DELTA:(k + 1) * C_DELTA],
             gks[k][:, D_POS:D_TAB]], axis=1))
            for k in range(K)]
        tb = b16(t)
        xt = [sum(sk[k] * tb[:, K * k + j:K * k + j + 1] for k in range(K))
              for j in range(K)]
        cvc16 = b16(cvc)
        ych = lbv
        for o in range(3):
            out_o = cvcb[o:o + 1, :]
            for j in range(K):
                out_o = out_o + b16(xt[j]) * cvc16[o * K + j:o * K + j + 1, :]
            ych = ych + dot(out_o, lwv[o * 50:(o + 1) * 50, :])
        bch = brow[:, sl]
        iot = jax.lax.broadcasted_iota(
            jnp.int32, (B, CH), 0).astype(jnp.float32)
        seg = (iot == bch).astype(jnp.float32)
        return (sums + xdot(seg, ych),
                cnt + jnp.sum(seg, axis=1, keepdims=True))

    sums, cnt = jax.lax.fori_loop(
        0, NCH, ph4,
        (jnp.zeros((B, 128), jnp.float32), jnp.zeros((B, 1), jnp.float32)))

    pooled = sums / jnp.maximum(cnt, 1.0)
    z = jnp.maximum(dot(pooled, c1_wT[...]) + c1_b[...], 0.0)
    s1, q1 = stats_pair(z)
    i1, h1 = finalize(s1, q1, B, bnc1_g, bnc1_b)
    z = z * i1 + h1
    z = jnp.maximum(dot(z, c2_wT[...]) + c2_b[...], 0.0)
    s2, q2 = stats_pair(z)
    i2, h2 = finalize(s2, q2, B, bnc2_g, bnc2_b)
    z = z * i2 + h2
    out[...] = 1.0 / (1.0 + jnp.exp(-z))


def kernel(x, pos, batch, num_graphs, l1_w, l1_b, bn1_g, bn1_b, l2_w, l2_b,
           bn2_g, bn2_b, l3_w, l3_b, bn3_g, bn3_b, cva_w, cva_b, bn4_g, bn4_b,
           cvb_w, cvb_b, bn5_g, bn5_b, cvc_w, cvc_b, lw, lb, c1_w, c1_b,
           bnc1_g, bnc1_b, c2_w, c2_b, bnc2_g, bnc2_b):
    del num_graphs
    table = jnp.concatenate(
        [pos, x, jnp.zeros((N, D_PAD - D_TAB), jnp.float32)], axis=1)
    posT = pos.T
    bf = batch.astype(jnp.float32)
    brow = bf.reshape(1, N)
    bcol = bf.reshape(N, 1)

    nblk = N // ROW_BLK
    idxf = pl.pallas_call(
        _knn_gather_kernel,
        grid=(nblk,),
        in_specs=[
            pl.BlockSpec((ROW_BLK, D_POS), lambda i: (i, 0)),
            pl.BlockSpec((D_POS, N), lambda i: (0, 0)),
            pl.BlockSpec((1, N), lambda i: (0, 0)),
            pl.BlockSpec((ROW_BLK, 1), lambda i: (i, 0)),
        ],
        out_specs=pl.BlockSpec((ROW_BLK, K), lambda i: (i, 0)),
        out_shape=jax.ShapeDtypeStruct((N, K), jnp.float32),
    )(pos, posT, brow, bcol)
    idx_flat = idxf.T.reshape(-1).astype(jnp.int32)
    g = _sc_gather(table, idx_flat)

    eye = jnp.eye(K, dtype=jnp.float32)
    wa = (jnp.einsum('gok,gh->gkho', cva_w, eye)).reshape(K * K, K * K)
    wb = (jnp.einsum('gok,gh->gkho', cvb_w, eye)).reshape(K * K, K * K)
    cvc_r = jnp.transpose(cvc_w, (1, 2, 0)).reshape(3 * K, 50)
    cvcb_r = cvc_b.reshape(50, 3).T
    lw_r = jnp.transpose(lw.reshape(128, 50, 3), (2, 1, 0)).reshape(150, 128)

    args = [
        g, pos, brow,
        l1_w.T, l1_b.reshape(1, -1), bn1_g.reshape(1, -1), bn1_b.reshape(1, -1),
        l2_w.T, l2_b.reshape(1, -1), bn2_g.reshape(1, -1), bn2_b.reshape(1, -1),
        l3_w.T, l3_b.reshape(1, -1), bn3_g.reshape(1, -1), bn3_b.reshape(1, -1),
        wa, cva_b.reshape(1, -1), bn4_g.reshape(1, -1), bn4_b.reshape(1, -1),
        wb, cvb_b.reshape(1, -1), bn5_g.reshape(1, -1), bn5_b.reshape(1, -1),
        cvc_r, cvcb_r, lw_r, lb.reshape(1, -1),
        c1_w.T, c1_b.reshape(1, -1), bnc1_g.reshape(1, -1), bnc1_b.reshape(1, -1),
        c2_w.T, c2_b.reshape(1, -1), bnc2_g.reshape(1, -1), bnc2_b.reshape(1, -1),
    ]
    out = pl.pallas_call(
        _mlp_kernel,
        out_shape=jax.ShapeDtypeStruct((B, 1), jnp.float32),
        scratch_shapes=[
            pltpu.VMEM((N, K * C_DELTA), jnp.float32),
            pltpu.VMEM((N, K * K), jnp.float32),
        ],
    )(*args)
    return out[:, 0]


# speedup vs baseline: 7.9755x; 1.0004x over previous
import functools

import jax
import jax.numpy as jnp
from jax.experimental import pallas as pl
from jax.experimental.pallas import tpu as pltpu
from jax.experimental.pallas import tpu_sc as plsc

N = 8192
B = 16
D_POS = 30
D_IN = 40
K = 4
C_DELTA = 10
ROW_BLK = 256
D_TAB = D_POS + D_IN


def _knn_gather_kernel(pos_blk_ref, posT, brow, bcol, idx_out):
    pos_blk = pos_blk_ref[...]
    posTv = posT[...]
    sqr = jnp.sum(pos_blk * pos_blk, axis=1, keepdims=True)
    sqc = jnp.sum(posTv * posTv, axis=0, keepdims=True)
    cross = jax.lax.dot_general(
        pos_blk.astype(jnp.bfloat16), posTv.astype(jnp.bfloat16),
        (((1,), (0,)), ((), ())),
        preferred_element_type=jnp.float32)
    d2 = sqr + sqc - 2.0 * cross
    mask = bcol[...] != brow[...]
    d2 = jnp.where(mask, 1e30, d2)

    iota = jax.lax.broadcasted_iota(jnp.int32, (ROW_BLK, N), 1)
    parts = []
    for _ in range(K):
        vmin = jnp.min(d2, axis=1, keepdims=True)
        idx = jnp.min(jnp.where(d2 == vmin, iota, N), axis=1, keepdims=True)
        parts.append(idx.astype(jnp.float32))
        d2 = jnp.where(iota == idx, 3e38, d2)
    idx_out[...] = jnp.concatenate(parts, axis=1)


D_PAD = 128
SC_CHUNK = 512


def _sc_gather(table_pad, idx_flat):
    info = plsc.get_sparse_core_info()
    nw = info.num_cores * info.num_subcores
    bpw = (K * N) // nw
    nchunk = bpw // SC_CHUNK
    mesh = plsc.VectorSubcoreMesh(core_axis_name="c", subcore_axis_name="s")

    @functools.partial(
        pl.kernel, mesh=mesh,
        out_type=jax.ShapeDtypeStruct((K * N, D_PAD), jnp.float32),
        scratch_types=[
            pltpu.VMEM((SC_CHUNK,), jnp.int32),
            pltpu.VMEM((SC_CHUNK, D_PAD), jnp.float32),
            pltpu.SemaphoreType.DMA,
        ],
    )
    def gk(table_hbm, idx_hbm, out_hbm, idx_v, rows_v, sem):
        wid = jax.lax.axis_index("s") * info.num_cores + jax.lax.axis_index("c")
        base = wid * bpw
        for c in range(nchunk):
            off = base + c * SC_CHUNK
            pltpu.sync_copy(idx_hbm.at[pl.ds(off, SC_CHUNK)], idx_v)
            pltpu.async_copy(table_hbm.at[idx_v], rows_v, sem).wait()
            pltpu.sync_copy(rows_v, out_hbm.at[pl.ds(off, SC_CHUNK)])

    return gk(table_pad, idx_flat)


def _elu(v):
    return jnp.where(v > 0, v, jnp.exp(jnp.minimum(v, 0.0)) - 1.0)


CH = 512
NCH = N // CH


def _mlp_kernel(g, pos, brow,
                l1_wT, l1_b, bn1_g, bn1_b, l2_wT, l2_b, bn2_g, bn2_b,
                l3_wT, l3_b, bn3_g, bn3_b, wa, cva_b, bn4_g, bn4_b,
                wb, cvb_b, bn5_g, bn5_b, cvc_r, cvcb_r, lw_r, lb,
                c1_wT, c1_b, bnc1_g, bnc1_b, c2_wT, c2_b, bnc2_g, bnc2_b,
                out, hs, ts):
    eps = 1e-5

    def dot(a, b):
        return jax.lax.dot_general(a.astype(jnp.bfloat16),
                                   b.astype(jnp.bfloat16),
                                   (((1,), (0,)), ((), ())),
                                   preferred_element_type=jnp.float32)

    def xdot(a, b):
        return jax.lax.dot_general(a, b, (((1,), (0,)), ((), ())),
                                   precision=jax.lax.Precision.HIGHEST,
                                   preferred_element_type=jnp.float32)

    def b16(v):
        return v.astype(jnp.bfloat16).astype(jnp.float32)

    l1w = l1_wT[...]; l1bv = l1_b[...]
    l2w = l2_wT[...]; l2bv = l2_b[...]
    l3w = l3_wT[...]; l3bv = l3_b[...]
    wav = wa[...]; cvabv = cva_b[...]
    wbv = wb[...]; cvbbv = cvb_b[...]
    cvc = cvc_r[...]; cvcb = cvcb_r[...]
    lwv = lw_r[...]; lbv = lb[...]

    def stats_pair(arr):
        return (jnp.sum(arr, axis=0, keepdims=True),
                jnp.sum(arr * arr, axis=0, keepdims=True))

    def finalize(s, q, rows, gam, bet):
        m = s / rows
        v = q / rows - m * m
        inv = gam[...] / jnp.sqrt(v + eps)
        return inv, bet[...] - m * inv

    def fold4(v40):
        return sum(v40[:, k * C_DELTA:(k + 1) * C_DELTA] for k in range(K))

    z40 = jnp.zeros((1, K * C_DELTA), jnp.float32)
    z16 = jnp.zeros((1, K * K), jnp.float32)

    def ph1(i, c):
        s40, q40, s16, q16 = c
        sl = pl.ds(i * CH, CH)
        pch = pos[sl, :]
        gks = [g[pl.ds(k * N + i * CH, CH), :] for k in range(K)]
        rel = [gk[:, :D_POS] - pch for gk in gks]
        hcat = jnp.concatenate(
            [_elu(dot(r, l1w) + l1bv) for r in rel], axis=1)
        t1 = _elu(dot(jnp.concatenate(rel, axis=1), l3w) + l3bv)
        hs[sl, :] = hcat
        ts[sl, :] = t1
        a, b2 = stats_pair(hcat)
        c2, d2 = stats_pair(t1)
        return (s40 + a, q40 + b2, s16 + c2, q16 + d2)

    s40, q40, s16, q16 = jax.lax.fori_loop(
        0, NCH, ph1, (z40, z40, z16, z16))
    inv1, sh1 = finalize(fold4(s40), fold4(q40), K * N, bn1_g, bn1_b)
    inv1 = jnp.concatenate([inv1] * K, axis=1)
    sh1 = jnp.concatenate([sh1] * K, axis=1)
    inv3, sh3 = finalize(s16, q16, N, bn3_g, bn3_b)

    def ph2(i, c):
        s40, q40, s16, q16 = c
        sl = pl.ds(i * CH, CH)
        h1 = hs[sl, :] * inv1 + sh1
        hcat = jnp.concatenate(
            [_elu(dot(h1[:, k * C_DELTA:(k + 1) * C_DELTA], l2w) + l2bv)
             for k in range(K)], axis=1)
        t1 = ts[sl, :] * inv3 + sh3
        t2 = _elu(dot(t1, wav) + cvabv)
        hs[sl, :] = hcat
        ts[sl, :] = t2
        a, b2 = stats_pair(hcat)
        c2, d2 = stats_pair(t2)
        return (s40 + a, q40 + b2, s16 + c2, q16 + d2)

    s40, q40, s16, q16 = jax.lax.fori_loop(
        0, NCH, ph2, (z40, z40, z16, z16))
    inv2, sh2 = finalize(fold4(s40), fold4(q40), K * N, bn2_g, bn2_b)
    inv2 = jnp.concatenate([inv2] * K, axis=1)
    sh2 = jnp.concatenate([sh2] * K, axis=1)
    inv4, sh4 = finalize(s16, q16, N, bn4_g, bn4_b)

    def ph3(i, c):
        s16, q16 = c
        sl = pl.ds(i * CH, CH)
        t2 = ts[sl, :] * inv4 + sh4
        t3 = dot(t2, wbv) + cvbbv
        ts[sl, :] = t3
        c2, d2 = stats_pair(t3)
        return (s16 + c2, q16 + d2)

    s16, q16 = jax.lax.fori_loop(0, NCH, ph3, (z16, z16))
    inv5, sh5 = finalize(s16, q16, N, bn5_g, bn5_b)

    def ph4(i, c):
        sums, cnt = c
        sl = pl.ds(i * CH, CH)
        hb = hs[sl, :] * inv2 + sh2
        t = ts[sl, :] * inv5 + sh5
        gks = [g[pl.ds(k * N + i * CH, CH), :] for k in range(K)]
        sk = [b16(jnp.concatenate(
            [hb[:, k * C_DELTA:(k + 1) * C_DELTA],
             gks[k][:, D_POS:D_TAB]], axis=1))
            for k in range(K)]
        tb = b16(t)
        xt = [sum(sk[k] * tb[:, K * k + j:K * k + j + 1] for k in range(K))
              for j in range(K)]
        cvc16 = b16(cvc)
        ych = lbv
        for o in range(3):
            out_o = cvcb[o:o + 1, :]
            for j in range(K):
                out_o = out_o + b16(xt[j]) * cvc16[o * K + j:o * K + j + 1, :]
            ych = ych + dot(out_o, lwv[o * 50:(o + 1) * 50, :])
        bch = brow[:, sl]
        iot = jax.lax.broadcasted_iota(
            jnp.int32, (B, CH), 0).astype(jnp.float32)
        seg = (iot == bch).astype(jnp.float32)
        return (sums + xdot(seg, ych),
                cnt + jnp.sum(seg, axis=1, keepdims=True))

    sums, cnt = jax.lax.fori_loop(
        0, NCH, ph4,
        (jnp.zeros((B, 128), jnp.float32), jnp.zeros((B, 1), jnp.float32)))

    pooled = sums / jnp.maximum(cnt, 1.0)
    z = jnp.maximum(dot(pooled, c1_wT[...]) + c1_b[...], 0.0)
    s1, q1 = stats_pair(z)
    i1, h1 = finalize(s1, q1, B, bnc1_g, bnc1_b)
    z = z * i1 + h1
    z = jnp.maximum(dot(z, c2_wT[...]) + c2_b[...], 0.0)
    s2, q2 = stats_pair(z)
    i2, h2 = finalize(s2, q2, B, bnc2_g, bnc2_b)
    z = z * i2 + h2
    out[...] = 1.0 / (1.0 + jnp.exp(-z))


def kernel(x, pos, batch, num_graphs, l1_w, l1_b, bn1_g, bn1_b, l2_w, l2_b,
           bn2_g, bn2_b, l3_w, l3_b, bn3_g, bn3_b, cva_w, cva_b, bn4_g, bn4_b,
           cvb_w, cvb_b, bn5_g, bn5_b, cvc_w, cvc_b, lw, lb, c1_w, c1_b,
           bnc1_g, bnc1_b, c2_w, c2_b, bnc2_g, bnc2_b):
    del num_graphs
    table = jnp.concatenate(
        [pos, x, jnp.zeros((N, D_PAD - D_TAB), jnp.float32)], axis=1)
    posT = pos.T
    bf = batch.astype(jnp.float32)
    brow = bf.reshape(1, N)
    bcol = bf.reshape(N, 1)

    nblk = N // ROW_BLK
    idxf = pl.pallas_call(
        _knn_gather_kernel,
        grid=(nblk,),
        in_specs=[
            pl.BlockSpec((ROW_BLK, D_POS), lambda i: (i, 0)),
            pl.BlockSpec((D_POS, N), lambda i: (0, 0)),
            pl.BlockSpec((1, N), lambda i: (0, 0)),
            pl.BlockSpec((ROW_BLK, 1), lambda i: (i, 0)),
        ],
        out_specs=pl.BlockSpec((ROW_BLK, K), lambda i: (i, 0)),
        out_shape=jax.ShapeDtypeStruct((N, K), jnp.float32),
        compiler_params=pltpu.CompilerParams(
            dimension_semantics=("parallel",)),
    )(pos, posT, brow, bcol)
    idx_flat = idxf.T.reshape(-1).astype(jnp.int32)
    g = _sc_gather(table, idx_flat)

    eye = jnp.eye(K, dtype=jnp.float32)
    wa = (jnp.einsum('gok,gh->gkho', cva_w, eye)).reshape(K * K, K * K)
    wb = (jnp.einsum('gok,gh->gkho', cvb_w, eye)).reshape(K * K, K * K)
    cvc_r = jnp.transpose(cvc_w, (1, 2, 0)).reshape(3 * K, 50)
    cvcb_r = cvc_b.reshape(50, 3).T
    lw_r = jnp.transpose(lw.reshape(128, 50, 3), (2, 1, 0)).reshape(150, 128)

    args = [
        g, pos, brow,
        l1_w.T, l1_b.reshape(1, -1), bn1_g.reshape(1, -1), bn1_b.reshape(1, -1),
        l2_w.T, l2_b.reshape(1, -1), bn2_g.reshape(1, -1), bn2_b.reshape(1, -1),
        l3_w.T, l3_b.reshape(1, -1), bn3_g.reshape(1, -1), bn3_b.reshape(1, -1),
        wa, cva_b.reshape(1, -1), bn4_g.reshape(1, -1), bn4_b.reshape(1, -1),
        wb, cvb_b.reshape(1, -1), bn5_g.reshape(1, -1), bn5_b.reshape(1, -1),
        cvc_r, cvcb_r, lw_r, lb.reshape(1, -1),
        c1_w.T, c1_b.reshape(1, -1), bnc1_g.reshape(1, -1), bnc1_b.reshape(1, -1),
        c2_w.T, c2_b.reshape(1, -1), bnc2_g.reshape(1, -1), bnc2_b.reshape(1, -1),
    ]
    out = pl.pallas_call(
        _mlp_kernel,
        out_shape=jax.ShapeDtypeStruct((B, 1), jnp.float32),
        scratch_shapes=[
            pltpu.VMEM((N, K * C_DELTA), jnp.float32),
            pltpu.VMEM((N, K * K), jnp.float32),
        ],
    )(*args)
    return out[:, 0]
